# baseline (device time: 38224 ns/iter reference)
import jax
import jax.numpy as jnp
from jax import lax
from jax.experimental import pallas as pl
from jax.experimental.pallas import tpu as pltpu

N_DEV = 32
N_STAGES = 5


def kernel(x, router_W, route_idx, expert_W):
    n, d = x.shape
    e_per, _, h = expert_W.shape

    def body(x_ref, route_ref, w_ref, out_ref, recv_buf, send_sems, recv_sems):
        my_pos = lax.axis_index("i")

        barrier_sem = pltpu.get_barrier_semaphore()
        for s in range(N_STAGES):
            partner = my_pos ^ (1 << s)
            pl.semaphore_signal(
                barrier_sem, inc=1,
                device_id=(partner,), device_id_type=pl.DeviceIdType.MESH,
            )
        pl.semaphore_wait(barrier_sem, N_STAGES)

        routes = route_ref[:, :]
        acc = jnp.zeros((n, h), jnp.float32)
        for e in range(e_per):
            eid = my_pos * e_per + e
            mask = (routes == eid).astype(jnp.float32)
            acc = acc + jnp.dot(
                x_ref[:, :] * mask, w_ref[e],
                preferred_element_type=jnp.float32,
            )
        out_ref[:, :] = acc

        for s in range(N_STAGES):
            partner = my_pos ^ (1 << s)
            rdma = pltpu.make_async_remote_copy(
                src_ref=out_ref,
                dst_ref=recv_buf.at[s],
                send_sem=send_sems.at[s],
                recv_sem=recv_sems.at[s],
                device_id=(partner,),
                device_id_type=pl.DeviceIdType.MESH,
            )
            rdma.start()
            rdma.wait()
            out_ref[:, :] = out_ref[:, :] + recv_buf[s]

    return pl.pallas_call(
        body,
        out_shape=jax.ShapeDtypeStruct((n, h), jnp.float32),
        in_specs=[
            pl.BlockSpec(memory_space=pltpu.VMEM),
            pl.BlockSpec(memory_space=pltpu.VMEM),
            pl.BlockSpec(memory_space=pltpu.VMEM),
        ],
        out_specs=pl.BlockSpec(memory_space=pltpu.VMEM),
        scratch_shapes=[
            pltpu.VMEM((N_STAGES, n, h), jnp.float32),
            pltpu.SemaphoreType.DMA((N_STAGES,)),
            pltpu.SemaphoreType.DMA((N_STAGES,)),
        ],
        compiler_params=pltpu.CompilerParams(collective_id=0),
    )(x, route_idx, expert_W)


# device time: 23249 ns/iter; 1.6441x vs baseline; 1.6441x over previous
import jax
import jax.numpy as jnp
from jax import lax
from jax.experimental import pallas as pl
from jax.experimental.pallas import tpu as pltpu

N_DEV = 32


def kernel(x, router_W, route_idx, expert_W):
    n, d = x.shape
    e_per, _, h = expert_W.shape
    ns8 = n // 8
    ns32 = n // 32

    def body(x_ref, route_ref, w_ref, out_ref,
             rbuf1, acc32, rbuf2, acc8, rbuf3,
             s1_send, s1_recv, s2_send, s2_recv,
             s3_send, s3_recv, s4_send, s4_recv):
        m = lax.axis_index("i")
        z = m // 8
        p = m % 8
        y = p // 2
        xr = p % 2
        xc = jnp.where(y % 2 == 0, xr, 1 - xr)

        def mesh_idx(xi, yi, zi):
            return zi * 8 + yi * 2 + jnp.where(yi % 2 == 0, xi, 1 - xi)

        r = xc * 4 + (y % 2) * 2 + (z % 2)
        q = (y // 2) * 2 + (z // 2)

        def cube_partner(j):
            xj = j // 4
            yj = 2 * (y // 2) + ((j // 2) % 2)
            zj = 2 * (z // 2) + (j % 2)
            return mesh_idx(xj, yj, zj)

        def far_partner(t):
            yt = 2 * (t // 2) + (y % 2)
            zt = 2 * (t % 2) + (z % 2)
            return mesh_idx(xc, yt, zt)

        barrier_sem = pltpu.get_barrier_semaphore()
        for j in range(8):
            @pl.when(r != j)
            def _(j=j):
                pl.semaphore_signal(
                    barrier_sem, inc=1,
                    device_id=(cube_partner(j),),
                    device_id_type=pl.DeviceIdType.MESH,
                )
        for t in range(4):
            @pl.when(q != t)
            def _(t=t):
                pl.semaphore_signal(
                    barrier_sem, inc=1,
                    device_id=(far_partner(t),),
                    device_id_type=pl.DeviceIdType.MESH,
                )
        pl.semaphore_wait(barrier_sem, 10)

        routes = route_ref[:, :]
        acc = jnp.zeros((n, h), jnp.float32)
        for e in range(e_per):
            eid = m * e_per + e
            mask = (routes == eid).astype(jnp.float32)
            acc = acc + jnp.dot(
                x_ref[:, :] * mask, w_ref[e],
                preferred_element_type=jnp.float32,
            )
        out_ref[:, :] = acc

        for j in range(8):
            @pl.when(r == j)
            def _(j=j):
                rbuf1[j] = out_ref[pl.ds(j * ns8, ns8), :]

            @pl.when(r != j)
            def _(j=j):
                pltpu.make_async_remote_copy(
                    src_ref=out_ref.at[pl.ds(j * ns8, ns8), :],
                    dst_ref=rbuf1.at[r],
                    send_sem=s1_send.at[j],
                    recv_sem=s1_recv.at[r],
                    device_id=(cube_partner(j),),
                    device_id_type=pl.DeviceIdType.MESH,
                ).start()
        for j in range(8):
            @pl.when(r != j)
            def _(j=j):
                pltpu.make_async_remote_copy(
                    src_ref=rbuf1.at[j],
                    dst_ref=rbuf1.at[j],
                    send_sem=s1_send.at[j],
                    recv_sem=s1_recv.at[j],
                    device_id=(m,),
                    device_id_type=pl.DeviceIdType.MESH,
                ).wait_recv()
        tot = rbuf1[0]
        for j in range(1, 8):
            tot = tot + rbuf1[j]
        acc32[:, :] = tot

        for t in range(4):
            @pl.when(q == t)
            def _(t=t):
                rbuf2[t] = acc32[pl.ds(t * ns32, ns32), :]

            @pl.when(q != t)
            def _(t=t):
                pltpu.make_async_remote_copy(
                    src_ref=acc32.at[pl.ds(t * ns32, ns32), :],
                    dst_ref=rbuf2.at[q],
                    send_sem=s2_send.at[t],
                    recv_sem=s2_recv.at[q],
                    device_id=(far_partner(t),),
                    device_id_type=pl.DeviceIdType.MESH,
                ).start()
        for t in range(4):
            @pl.when(q != t)
            def _(t=t):
                pltpu.make_async_remote_copy(
                    src_ref=rbuf2.at[t],
                    dst_ref=rbuf2.at[t],
                    send_sem=s2_send.at[t],
                    recv_sem=s2_recv.at[t],
                    device_id=(m,),
                    device_id_type=pl.DeviceIdType.MESH,
                ).wait_recv()
        acc8[:, :] = rbuf2[0] + rbuf2[1] + rbuf2[2] + rbuf2[3]

        for t in range(4):
            @pl.when(q != t)
            def _(t=t):
                pltpu.make_async_remote_copy(
                    src_ref=acc8,
                    dst_ref=rbuf3.at[q],
                    send_sem=s3_send.at[t],
                    recv_sem=s3_recv.at[q],
                    device_id=(far_partner(t),),
                    device_id_type=pl.DeviceIdType.MESH,
                ).start()
        for t in range(4):
            @pl.when(q == t)
            def _(t=t):
                acc32[pl.ds(t * ns32, ns32), :] = acc8[:, :]

            @pl.when(q != t)
            def _(t=t):
                pltpu.make_async_remote_copy(
                    src_ref=rbuf3.at[t],
                    dst_ref=rbuf3.at[t],
                    send_sem=s3_send.at[t],
                    recv_sem=s3_recv.at[t],
                    device_id=(m,),
                    device_id_type=pl.DeviceIdType.MESH,
                ).wait_recv()
                acc32[pl.ds(t * ns32, ns32), :] = rbuf3[t]

        for j in range(8):
            @pl.when(r == j)
            def _(j=j):
                out_ref[pl.ds(j * ns8, ns8), :] = acc32[:, :]

            @pl.when(r != j)
            def _(j=j):
                pltpu.make_async_remote_copy(
                    src_ref=acc32,
                    dst_ref=out_ref.at[pl.ds(r * ns8, ns8), :],
                    send_sem=s4_send.at[j],
                    recv_sem=s4_recv.at[r],
                    device_id=(cube_partner(j),),
                    device_id_type=pl.DeviceIdType.MESH,
                ).start()
        for j in range(8):
            @pl.when(r != j)
            def _(j=j):
                pltpu.make_async_remote_copy(
                    src_ref=out_ref.at[pl.ds(j * ns8, ns8), :],
                    dst_ref=out_ref.at[pl.ds(j * ns8, ns8), :],
                    send_sem=s4_send.at[j],
                    recv_sem=s4_recv.at[j],
                    device_id=(m,),
                    device_id_type=pl.DeviceIdType.MESH,
                ).wait_recv()

        for j in range(8):
            @pl.when(r != j)
            def _(j=j):
                pltpu.make_async_remote_copy(
                    src_ref=out_ref.at[pl.ds(j * ns8, ns8), :],
                    dst_ref=out_ref.at[pl.ds(j * ns8, ns8), :],
                    send_sem=s1_send.at[j],
                    recv_sem=s1_recv.at[j],
                    device_id=(m,),
                    device_id_type=pl.DeviceIdType.MESH,
                ).wait_send()
                pltpu.make_async_remote_copy(
                    src_ref=acc32,
                    dst_ref=out_ref.at[pl.ds(j * ns8, ns8), :],
                    send_sem=s4_send.at[j],
                    recv_sem=s4_recv.at[j],
                    device_id=(m,),
                    device_id_type=pl.DeviceIdType.MESH,
                ).wait_send()
        for t in range(4):
            @pl.when(q != t)
            def _(t=t):
                pltpu.make_async_remote_copy(
                    src_ref=acc32.at[pl.ds(t * ns32, ns32), :],
                    dst_ref=rbuf2.at[t],
                    send_sem=s2_send.at[t],
                    recv_sem=s2_recv.at[t],
                    device_id=(m,),
                    device_id_type=pl.DeviceIdType.MESH,
                ).wait_send()
                pltpu.make_async_remote_copy(
                    src_ref=acc8,
                    dst_ref=rbuf3.at[t],
                    send_sem=s3_send.at[t],
                    recv_sem=s3_recv.at[t],
                    device_id=(m,),
                    device_id_type=pl.DeviceIdType.MESH,
                ).wait_send()

    return pl.pallas_call(
        body,
        out_shape=jax.ShapeDtypeStruct((n, h), jnp.float32),
        in_specs=[
            pl.BlockSpec(memory_space=pltpu.VMEM),
            pl.BlockSpec(memory_space=pltpu.VMEM),
            pl.BlockSpec(memory_space=pltpu.VMEM),
        ],
        out_specs=pl.BlockSpec(memory_space=pltpu.VMEM),
        scratch_shapes=[
            pltpu.VMEM((8, ns8, h), jnp.float32),
            pltpu.VMEM((ns8, h), jnp.float32),
            pltpu.VMEM((4, ns32, h), jnp.float32),
            pltpu.VMEM((ns32, h), jnp.float32),
            pltpu.VMEM((4, ns32, h), jnp.float32),
            pltpu.SemaphoreType.DMA((8,)),
            pltpu.SemaphoreType.DMA((8,)),
            pltpu.SemaphoreType.DMA((4,)),
            pltpu.SemaphoreType.DMA((4,)),
            pltpu.SemaphoreType.DMA((4,)),
            pltpu.SemaphoreType.DMA((4,)),
            pltpu.SemaphoreType.DMA((8,)),
            pltpu.SemaphoreType.DMA((8,)),
        ],
        compiler_params=pltpu.CompilerParams(collective_id=0),
    )(x, route_idx, expert_W)
